# baseline (speedup 1.0000x reference)
import functools

import jax
import jax.numpy as jnp
from jax.experimental import pallas as pl
from jax.experimental.pallas import tpu as pltpu

_BN_EPS = 1e-5
_BPB = 4

_DN = (((1,), (1,)), ((), ()))
_DN0 = (((0,), (0,)), ((), ()))


def _fused_kernel(x_ref, y_ref, w1_ref, w2_ref, g1_ref, bt1_ref,
                  g2_ref, bt2_ref, o_ref,
                  h1_scr, h2_scr, acc_scr, aff_scr, *, n_blk, inv_m):
    i = pl.program_id(0)

    @pl.when(i == 0)
    def _init():
        acc_scr[...] = jnp.zeros_like(acc_scr)

    @pl.when(i < n_blk)
    def _matmul_phase():
        w1b = w1_ref[...].astype(jnp.bfloat16)
        w2b = w2_ref[...].astype(jnp.bfloat16)
        c = w1_ref.shape[0]
        eye = jnp.eye(c, dtype=jnp.float32)
        for b in range(_BPB):
            h1 = jax.lax.dot_general(x_ref[b], w1b, _DN,
                                     preferred_element_type=jnp.float32)
            h2 = jax.lax.dot_general(y_ref[b], w2b, _DN,
                                     preferred_element_type=jnp.float32)
            h1b = h1.astype(h1_scr.dtype)
            h2b = h2.astype(h2_scr.dtype)
            h1_scr[i * _BPB + b] = h1b
            h2_scr[i * _BPB + b] = h2b
            mb = h1b.shape[0]
            ones = jnp.ones((1, mb), dtype=jnp.bfloat16)
            s1 = jnp.dot(ones, h1b, preferred_element_type=jnp.float32)
            s2 = jnp.dot(ones, h2b, preferred_element_type=jnp.float32)
            hh1 = jax.lax.dot_general(h1b, h1b, _DN0,
                                      preferred_element_type=jnp.float32)
            hh2 = jax.lax.dot_general(h2b, h2b, _DN0,
                                      preferred_element_type=jnp.float32)
            q1 = jnp.sum(hh1 * eye, axis=0, keepdims=True)
            q2 = jnp.sum(hh2 * eye, axis=0, keepdims=True)
            acc_scr[...] += jnp.concatenate([s1, s2, q1, q2], axis=0)

    @pl.when(i == n_blk)
    def _fold():
        acc = acc_scr[...]
        mean = acc[0:2] * inv_m
        var = acc[2:4] * inv_m - mean * mean
        gamma = jnp.concatenate([g1_ref[...], g2_ref[...]], axis=0)
        beta = jnp.concatenate([bt1_ref[...], bt2_ref[...]], axis=0)
        scale = gamma * jax.lax.rsqrt(var + _BN_EPS)
        shift = beta - scale * mean
        aff_scr[...] = jnp.concatenate([scale, shift], axis=0)

    @pl.when(i >= n_blk)
    def _apply_phase():
        j = i - n_blk
        aff = aff_scr[...]
        sc1 = aff[0:1]
        sc2 = aff[1:2]
        sh1 = aff[2:3]
        sh2 = aff[3:4]
        for b in range(_BPB):
            z1 = jnp.maximum(
                sc1 * h1_scr[j * _BPB + b].astype(jnp.float32) + sh1, 0.0)
            z2 = jnp.maximum(
                sc2 * h2_scr[j * _BPB + b].astype(jnp.float32) + sh2, 0.0)
            g = jax.nn.sigmoid(z1 + z2)
            o_ref[b] = (z2 + g * (z1 - z2)).astype(o_ref.dtype)


def kernel(x, y, w1, b1, w2, b2, gamma1, beta1, gamma2, beta2):
    n, c, hh, ww = x.shape
    hw = hh * ww
    m = n * hw
    out_dtype = x.dtype

    xt = jnp.transpose(x, (0, 2, 3, 1)).astype(jnp.bfloat16).reshape(n, hw, c)
    yt = jnp.transpose(y, (0, 2, 3, 1)).astype(jnp.bfloat16).reshape(n, hw, c)
    g1 = gamma1.reshape(1, c).astype(jnp.float32)
    bt1 = beta1.reshape(1, c).astype(jnp.float32)
    g2 = gamma2.reshape(1, c).astype(jnp.float32)
    bt2 = beta2.reshape(1, c).astype(jnp.float32)

    nb = n // _BPB
    last = nb - 1
    in_spec = pl.BlockSpec((_BPB, hw, c),
                           lambda i: (jnp.minimum(i, last), 0, 0))
    w_spec = pl.BlockSpec((c, c), lambda i: (0, 0))
    vec_spec = pl.BlockSpec((1, c), lambda i: (0, 0))
    out_spec = pl.BlockSpec((_BPB, hw, c),
                            lambda i: (jnp.maximum(i - nb, 0), 0, 0))

    body = functools.partial(_fused_kernel, n_blk=nb, inv_m=1.0 / m)
    out_t = pl.pallas_call(
        body,
        grid=(2 * nb,),
        in_specs=[in_spec, in_spec, w_spec, w_spec,
                  vec_spec, vec_spec, vec_spec, vec_spec],
        out_specs=out_spec,
        out_shape=jax.ShapeDtypeStruct((n, hw, c), jnp.bfloat16),
        scratch_shapes=[
            pltpu.VMEM((n, hw, c), jnp.bfloat16),
            pltpu.VMEM((n, hw, c), jnp.bfloat16),
            pltpu.VMEM((4, c), jnp.float32),
            pltpu.VMEM((4, c), jnp.float32),
        ],
        compiler_params=pltpu.CompilerParams(
            dimension_semantics=("arbitrary",),
            vmem_limit_bytes=60 * 1024 * 1024),
    )(xt, yt, w1, w2, g1, bt1, g2, bt2)

    out4 = out_t.reshape(n, hh, ww, c)
    return jnp.transpose(out4, (0, 3, 1, 2)).astype(out_dtype)


# speedup vs baseline: 1.1031x; 1.1031x over previous
import functools

import jax
import jax.numpy as jnp
from jax.experimental import pallas as pl
from jax.experimental.pallas import tpu as pltpu

_BN_EPS = 1e-5
_BPB = 2

_DN = (((1,), (1,)), ((), ()))
_DN0 = (((0,), (0,)), ((), ()))


def _fused_kernel(x_ref, y_ref, w1_ref, w2_ref, g1_ref, bt1_ref,
                  g2_ref, bt2_ref, o_ref,
                  h1_scr, h2_scr, acc_scr, aff_scr, *, n_blk, inv_m):
    i = pl.program_id(0)

    @pl.when(i == 0)
    def _init():
        acc_scr[...] = jnp.zeros_like(acc_scr)

    @pl.when(i < n_blk)
    def _matmul_phase():
        w1b = w1_ref[...].astype(jnp.bfloat16)
        w2b = w2_ref[...].astype(jnp.bfloat16)
        for b in range(_BPB):
            h1 = jax.lax.dot_general(x_ref[b], w1b, _DN,
                                     preferred_element_type=jnp.float32)
            h2 = jax.lax.dot_general(y_ref[b], w2b, _DN,
                                     preferred_element_type=jnp.float32)
            h1_scr[i * _BPB + b] = h1.astype(h1_scr.dtype)
            h2_scr[i * _BPB + b] = h2.astype(h2_scr.dtype)
            acc_scr[...] += jnp.concatenate(
                [jnp.sum(h1, axis=0, keepdims=True),
                 jnp.sum(h2, axis=0, keepdims=True),
                 jnp.sum(h1 * h1, axis=0, keepdims=True),
                 jnp.sum(h2 * h2, axis=0, keepdims=True)], axis=0)

    @pl.when(i == n_blk)
    def _fold():
        acc = acc_scr[...]
        mean = acc[0:2] * inv_m
        var = acc[2:4] * inv_m - mean * mean
        gamma = jnp.concatenate([g1_ref[...], g2_ref[...]], axis=0)
        beta = jnp.concatenate([bt1_ref[...], bt2_ref[...]], axis=0)
        scale = gamma * jax.lax.rsqrt(var + _BN_EPS)
        shift = beta - scale * mean
        aff_scr[...] = jnp.concatenate([scale, shift], axis=0)

    @pl.when(i >= n_blk)
    def _apply_phase():
        j = i - n_blk
        aff = aff_scr[...]
        sc1 = aff[0:1]
        sc2 = aff[1:2]
        sh1 = aff[2:3]
        sh2 = aff[3:4]
        for b in range(_BPB):
            z1 = jnp.maximum(
                sc1 * h1_scr[j * _BPB + b].astype(jnp.float32) + sh1, 0.0)
            z2 = jnp.maximum(
                sc2 * h2_scr[j * _BPB + b].astype(jnp.float32) + sh2, 0.0)
            g = jax.nn.sigmoid(z1 + z2)
            o_ref[b] = (z2 + g * (z1 - z2)).astype(o_ref.dtype)


def kernel(x, y, w1, b1, w2, b2, gamma1, beta1, gamma2, beta2):
    n, c, hh, ww = x.shape
    hw = hh * ww
    m = n * hw
    out_dtype = x.dtype

    xt = jnp.transpose(x, (0, 2, 3, 1)).astype(jnp.bfloat16).reshape(n, hw, c)
    yt = jnp.transpose(y, (0, 2, 3, 1)).astype(jnp.bfloat16).reshape(n, hw, c)
    g1 = gamma1.reshape(1, c).astype(jnp.float32)
    bt1 = beta1.reshape(1, c).astype(jnp.float32)
    g2 = gamma2.reshape(1, c).astype(jnp.float32)
    bt2 = beta2.reshape(1, c).astype(jnp.float32)

    nb = n // _BPB
    last = nb - 1
    in_spec = pl.BlockSpec((_BPB, hw, c),
                           lambda i: (jnp.minimum(i, last), 0, 0))
    w_spec = pl.BlockSpec((c, c), lambda i: (0, 0))
    vec_spec = pl.BlockSpec((1, c), lambda i: (0, 0))
    out_spec = pl.BlockSpec((_BPB, hw, c),
                            lambda i: (jnp.maximum(i - nb, 0), 0, 0))

    body = functools.partial(_fused_kernel, n_blk=nb, inv_m=1.0 / m)
    out_t = pl.pallas_call(
        body,
        grid=(2 * nb,),
        in_specs=[in_spec, in_spec, w_spec, w_spec,
                  vec_spec, vec_spec, vec_spec, vec_spec],
        out_specs=out_spec,
        out_shape=jax.ShapeDtypeStruct((n, hw, c), jnp.bfloat16),
        scratch_shapes=[
            pltpu.VMEM((n, hw, c), jnp.bfloat16),
            pltpu.VMEM((n, hw, c), jnp.bfloat16),
            pltpu.VMEM((4, c), jnp.float32),
            pltpu.VMEM((4, c), jnp.float32),
        ],
        compiler_params=pltpu.CompilerParams(
            dimension_semantics=("arbitrary",),
            vmem_limit_bytes=60 * 1024 * 1024),
    )(xt, yt, w1, w2, g1, bt1, g2, bt2)

    out4 = out_t.reshape(n, hh, ww, c)
    return jnp.transpose(out4, (0, 3, 1, 2)).astype(out_dtype)


# speedup vs baseline: 1.1423x; 1.0355x over previous
import functools

import jax
import jax.numpy as jnp
from jax.experimental import pallas as pl
from jax.experimental.pallas import tpu as pltpu

_BN_EPS = 1e-5
_BPB = 8

_DN = (((1,), (1,)), ((), ()))
_DN0 = (((0,), (0,)), ((), ()))


def _fused_kernel(x_ref, y_ref, w1_ref, w2_ref, g1_ref, bt1_ref,
                  g2_ref, bt2_ref, o_ref,
                  h1_scr, h2_scr, acc_scr, aff_scr, *, n_blk, inv_m):
    i = pl.program_id(0)

    @pl.when(i == 0)
    def _init():
        acc_scr[...] = jnp.zeros_like(acc_scr)

    @pl.when(i < n_blk)
    def _matmul_phase():
        w1b = w1_ref[...].astype(jnp.bfloat16)
        w2b = w2_ref[...].astype(jnp.bfloat16)
        for b in range(_BPB):
            h1 = jax.lax.dot_general(x_ref[b], w1b, _DN,
                                     preferred_element_type=jnp.float32)
            h2 = jax.lax.dot_general(y_ref[b], w2b, _DN,
                                     preferred_element_type=jnp.float32)
            h1_scr[i * _BPB + b] = h1.astype(h1_scr.dtype)
            h2_scr[i * _BPB + b] = h2.astype(h2_scr.dtype)
            acc_scr[...] += jnp.concatenate(
                [jnp.sum(h1, axis=0, keepdims=True),
                 jnp.sum(h2, axis=0, keepdims=True),
                 jnp.sum(h1 * h1, axis=0, keepdims=True),
                 jnp.sum(h2 * h2, axis=0, keepdims=True)], axis=0)

    @pl.when(i == n_blk)
    def _fold():
        acc = acc_scr[...]
        mean = acc[0:2] * inv_m
        var = acc[2:4] * inv_m - mean * mean
        gamma = jnp.concatenate([g1_ref[...], g2_ref[...]], axis=0)
        beta = jnp.concatenate([bt1_ref[...], bt2_ref[...]], axis=0)
        scale = gamma * jax.lax.rsqrt(var + _BN_EPS)
        shift = beta - scale * mean
        aff_scr[...] = jnp.concatenate([scale, shift], axis=0)

    @pl.when(i >= n_blk)
    def _apply_phase():
        j = i - n_blk
        aff = aff_scr[...]
        sc1 = aff[0:1]
        sc2 = aff[1:2]
        sh1 = aff[2:3]
        sh2 = aff[3:4]
        for b in range(_BPB):
            z1 = jnp.maximum(
                sc1 * h1_scr[j * _BPB + b].astype(jnp.float32) + sh1, 0.0)
            z2 = jnp.maximum(
                sc2 * h2_scr[j * _BPB + b].astype(jnp.float32) + sh2, 0.0)
            g = jax.nn.sigmoid(z1 + z2)
            o_ref[b] = (z2 + g * (z1 - z2)).astype(o_ref.dtype)


def kernel(x, y, w1, b1, w2, b2, gamma1, beta1, gamma2, beta2):
    n, c, hh, ww = x.shape
    hw = hh * ww
    m = n * hw
    out_dtype = x.dtype

    xt = jnp.transpose(x, (0, 2, 3, 1)).astype(jnp.bfloat16).reshape(n, hw, c)
    yt = jnp.transpose(y, (0, 2, 3, 1)).astype(jnp.bfloat16).reshape(n, hw, c)
    g1 = gamma1.reshape(1, c).astype(jnp.float32)
    bt1 = beta1.reshape(1, c).astype(jnp.float32)
    g2 = gamma2.reshape(1, c).astype(jnp.float32)
    bt2 = beta2.reshape(1, c).astype(jnp.float32)

    nb = n // _BPB
    last = nb - 1
    in_spec = pl.BlockSpec((_BPB, hw, c),
                           lambda i: (jnp.minimum(i, last), 0, 0))
    w_spec = pl.BlockSpec((c, c), lambda i: (0, 0))
    vec_spec = pl.BlockSpec((1, c), lambda i: (0, 0))
    out_spec = pl.BlockSpec((_BPB, hw, c),
                            lambda i: (jnp.maximum(i - nb, 0), 0, 0))

    body = functools.partial(_fused_kernel, n_blk=nb, inv_m=1.0 / m)
    out_t = pl.pallas_call(
        body,
        grid=(2 * nb,),
        in_specs=[in_spec, in_spec, w_spec, w_spec,
                  vec_spec, vec_spec, vec_spec, vec_spec],
        out_specs=out_spec,
        out_shape=jax.ShapeDtypeStruct((n, hw, c), jnp.bfloat16),
        scratch_shapes=[
            pltpu.VMEM((n, hw, c), jnp.bfloat16),
            pltpu.VMEM((n, hw, c), jnp.bfloat16),
            pltpu.VMEM((4, c), jnp.float32),
            pltpu.VMEM((4, c), jnp.float32),
        ],
        compiler_params=pltpu.CompilerParams(
            dimension_semantics=("arbitrary",),
            vmem_limit_bytes=60 * 1024 * 1024),
    )(xt, yt, w1, w2, g1, bt1, g2, bt2)

    out4 = out_t.reshape(n, hh, ww, c)
    return jnp.transpose(out4, (0, 3, 1, 2)).astype(out_dtype)


# speedup vs baseline: 1.1756x; 1.0292x over previous
import functools

import jax
import jax.numpy as jnp
from jax.experimental import pallas as pl
from jax.experimental.pallas import tpu as pltpu

_BN_EPS = 1e-5
_BPB = 4

_DN = (((1,), (1,)), ((), ()))
_DN0 = (((0,), (0,)), ((), ()))


def _fused_kernel(x_ref, y_ref, w1_ref, w2_ref, g1_ref, bt1_ref,
                  g2_ref, bt2_ref, o_ref,
                  h1_scr, h2_scr, acc_scr, aff_scr, *, n_blk, inv_m):
    i = pl.program_id(0)

    @pl.when(i == 0)
    def _init():
        acc_scr[...] = jnp.zeros_like(acc_scr)

    @pl.when(i < n_blk)
    def _matmul_phase():
        w1b = w1_ref[...].astype(jnp.bfloat16)
        w2b = w2_ref[...].astype(jnp.bfloat16)
        for b in range(_BPB):
            h1 = jax.lax.dot_general(x_ref[b], w1b, _DN,
                                     preferred_element_type=jnp.float32)
            h2 = jax.lax.dot_general(y_ref[b], w2b, _DN,
                                     preferred_element_type=jnp.float32)
            h1_scr[i * _BPB + b] = h1.astype(h1_scr.dtype)
            h2_scr[i * _BPB + b] = h2.astype(h2_scr.dtype)
            acc_scr[...] += jnp.concatenate(
                [jnp.sum(h1, axis=0, keepdims=True),
                 jnp.sum(h2, axis=0, keepdims=True),
                 jnp.sum(h1 * h1, axis=0, keepdims=True),
                 jnp.sum(h2 * h2, axis=0, keepdims=True)], axis=0)

    @pl.when(i == n_blk)
    def _fold():
        acc = acc_scr[...]
        mean = acc[0:2] * inv_m
        var = acc[2:4] * inv_m - mean * mean
        gamma = jnp.concatenate([g1_ref[...], g2_ref[...]], axis=0)
        beta = jnp.concatenate([bt1_ref[...], bt2_ref[...]], axis=0)
        scale = gamma * jax.lax.rsqrt(var + _BN_EPS)
        shift = beta - scale * mean
        aff_scr[...] = jnp.concatenate([scale, shift], axis=0)

    @pl.when(i >= n_blk)
    def _apply_phase():
        j = i - n_blk
        aff = aff_scr[...]
        sc1 = aff[0:1]
        sc2 = aff[1:2]
        sh1 = aff[2:3]
        sh2 = aff[3:4]
        for b in range(_BPB):
            z1 = jnp.maximum(
                sc1 * h1_scr[j * _BPB + b].astype(jnp.float32) + sh1, 0.0)
            z2 = jnp.maximum(
                sc2 * h2_scr[j * _BPB + b].astype(jnp.float32) + sh2, 0.0)
            g = jax.nn.sigmoid(z1 + z2)
            o_ref[b] = (z2 + g * (z1 - z2)).astype(o_ref.dtype)


def kernel(x, y, w1, b1, w2, b2, gamma1, beta1, gamma2, beta2):
    n, c, hh, ww = x.shape
    hw = hh * ww
    m = n * hw
    out_dtype = x.dtype

    xt = jnp.transpose(x, (0, 2, 3, 1)).astype(jnp.bfloat16).reshape(n, hw, c)
    yt = jnp.transpose(y, (0, 2, 3, 1)).astype(jnp.bfloat16).reshape(n, hw, c)
    g1 = gamma1.reshape(1, c).astype(jnp.float32)
    bt1 = beta1.reshape(1, c).astype(jnp.float32)
    g2 = gamma2.reshape(1, c).astype(jnp.float32)
    bt2 = beta2.reshape(1, c).astype(jnp.float32)

    nb = n // _BPB
    last = nb - 1
    in_spec = pl.BlockSpec((_BPB, hw, c),
                           lambda i: (jnp.minimum(i, last), 0, 0))
    w_spec = pl.BlockSpec((c, c), lambda i: (0, 0))
    vec_spec = pl.BlockSpec((1, c), lambda i: (0, 0))
    out_spec = pl.BlockSpec((_BPB, hw, c),
                            lambda i: (jnp.maximum(i - nb, 0), 0, 0))

    body = functools.partial(_fused_kernel, n_blk=nb, inv_m=1.0 / m)
    out_t = pl.pallas_call(
        body,
        grid=(2 * nb,),
        in_specs=[in_spec, in_spec, w_spec, w_spec,
                  vec_spec, vec_spec, vec_spec, vec_spec],
        out_specs=out_spec,
        out_shape=jax.ShapeDtypeStruct((n, hw, c), jnp.bfloat16),
        scratch_shapes=[
            pltpu.VMEM((n, hw, c), jnp.float32),
            pltpu.VMEM((n, hw, c), jnp.float32),
            pltpu.VMEM((4, c), jnp.float32),
            pltpu.VMEM((4, c), jnp.float32),
        ],
        compiler_params=pltpu.CompilerParams(
            dimension_semantics=("arbitrary",),
            vmem_limit_bytes=60 * 1024 * 1024),
    )(xt, yt, w1, w2, g1, bt1, g2, bt2)

    out4 = out_t.reshape(n, hh, ww, c)
    return jnp.transpose(out4, (0, 3, 1, 2)).astype(out_dtype)
